# baseline (speedup 1.0000x reference)
import functools
import math

import jax
import jax.numpy as jnp
import numpy as np
from jax import lax
from jax.experimental import pallas as pl
from jax.experimental.pallas import tpu as pltpu
from jax.experimental.pallas import tpu_sc as plsc

HIDDEN = 128
VOCAB = 30522
MAX_SEQ = 512
BATCH = 1024

N_TOK = BATCH * MAX_SEQ
NUM_WORKERS = 32
TOK_PER_W = N_TOK // NUM_WORKERS
CHUNK = 64
NCHUNK = TOK_PER_W // CHUNK
NBUF = 4
NSUPER = NCHUNK // NBUF
POS_PERIOD = MAX_SEQ // CHUNK
LANES = 16
PACKED = HIDDEN // 2
NBLK = HIDDEN // (2 * LANES)
SCALE = math.sqrt(HIDDEN)


def _pos_encoding(max_seq_len, hidden):
    pe = np.zeros((max_seq_len, hidden), dtype=np.float32)
    pos = np.arange(max_seq_len, dtype=np.float64)[:, None]
    i = np.arange(0, hidden, 2, dtype=np.float64)
    pe[:, 0::2] = np.sin(pos / (10000.0 ** (2.0 * i / hidden)))
    pe[:, 1::2] = np.cos(pos / (10000.0 ** (2.0 * (i + 1.0) / hidden)))
    return pe


_PE = _pos_encoding(MAX_SEQ, HIDDEN)


def _pack_bf16(x):
    n = x.shape[0]
    xb = x.reshape(n, NBLK, 2, LANES)
    a = lax.bitcast_convert_type(
        xb[:, :, 0, :].astype(jnp.bfloat16), jnp.uint16
    ).astype(jnp.uint32)
    b = lax.bitcast_convert_type(
        xb[:, :, 1, :].astype(jnp.bfloat16), jnp.uint16
    ).astype(jnp.uint32)
    return lax.bitcast_convert_type(a | (b << 16), jnp.int32).reshape(n, PACKED)


@functools.partial(
    pl.kernel,
    out_type=jax.ShapeDtypeStruct((N_TOK, HIDDEN), jnp.float32),
    mesh=plsc.VectorSubcoreMesh(core_axis_name="c", subcore_axis_name="s"),
    compiler_params=pltpu.CompilerParams(
        needs_layout_passes=False, use_tc_tiling_on_sc=False
    ),
    scratch_types=[
        pltpu.VMEM((TOK_PER_W,), jnp.int32),
        pltpu.VMEM((MAX_SEQ, PACKED), jnp.int32),
        pltpu.VMEM((NBUF, CHUNK, PACKED), jnp.int32),
        pltpu.VMEM((NBUF, CHUNK, HIDDEN), jnp.float32),
        pltpu.SemaphoreType.DMA((NBUF,)),
        pltpu.SemaphoreType.DMA((NBUF,)),
    ],
)
def _encode(idx_hbm, tbl_hbm, pe_hbm, out_hbm,
            idx_v, pe_v, gath_v, out_v, gsem, wsem):
    wid = lax.axis_index("s") * 2 + lax.axis_index("c")
    base = wid * TOK_PER_W
    pltpu.sync_copy(pe_hbm, pe_v)
    pltpu.sync_copy(idx_hbm.at[pl.ds(base, TOK_PER_W)], idx_v)

    def start_gather(b, c):
        pltpu.async_copy(
            tbl_hbm.at[idx_v.at[pl.ds(c * CHUNK, CHUNK)]],
            gath_v.at[b],
            gsem.at[b],
        )

    def wait_gather(b):
        pltpu.make_async_copy(
            tbl_hbm.at[idx_v.at[pl.ds(0, CHUNK)]], gath_v.at[b], gsem.at[b]
        ).wait()

    def start_write(b, c):
        pltpu.async_copy(
            out_v.at[b], out_hbm.at[pl.ds(base + c * CHUNK, CHUNK)], wsem.at[b]
        )

    def wait_write(b):
        pltpu.make_async_copy(
            out_v.at[b], out_hbm.at[pl.ds(base, CHUNK)], wsem.at[b]
        ).wait()

    def compute(b, c):
        gbuf = gath_v.at[b]
        obuf = out_v.at[b]
        prow = (c % POS_PERIOD) * CHUNK

        @plsc.parallel_loop(0, CHUNK, 1, unroll=4)
        def _(j):
            for k in range(NBLK):
                sl = pl.ds(k * LANES, LANES)
                u = plsc.bitcast(gbuf[j, sl], jnp.bfloat16)
                p = plsc.bitcast(pe_v[prow + j, sl], jnp.bfloat16)
                r_lo, r_hi = plsc.unpack(u, format=plsc.PackFormat.INTERLEAVED)
                p_lo, p_hi = plsc.unpack(p, format=plsc.PackFormat.INTERLEAVED)
                obuf[j, pl.ds(2 * k * LANES, LANES)] = r_lo * SCALE + p_lo
                obuf[j, pl.ds((2 * k + 1) * LANES, LANES)] = r_hi * SCALE + p_hi

    for b in range(NBUF):
        start_gather(b, b)

    for b in range(NBUF):
        wait_gather(b)
        compute(b, b)
        start_gather(b, NBUF + b)
        start_write(b, b)

    def super_step(s, carry):
        for b in range(NBUF):
            c = s * NBUF + b
            wait_gather(b)
            wait_write(b)
            compute(b, c)
            start_gather(b, c + NBUF)
            start_write(b, c)
        return carry

    lax.fori_loop(1, NSUPER - 1, super_step, 0)

    for b in range(NBUF):
        c = (NSUPER - 1) * NBUF + b
        wait_gather(b)
        wait_write(b)
        compute(b, c)
        start_write(b, c)
    for b in range(NBUF):
        wait_write(b)


def kernel(text_batch, embed_table):
    b, l = text_batch.shape
    idx = text_batch.reshape(-1)
    tbl = _pack_bf16(embed_table)
    pe = _pack_bf16(jnp.asarray(_PE))
    out = _encode(idx, tbl, pe)
    return out.reshape(b, l, HIDDEN)


# speedup vs baseline: 1.0707x; 1.0707x over previous
import functools
import math

import jax
import jax.numpy as jnp
import numpy as np
from jax import lax
from jax.experimental import pallas as pl
from jax.experimental.pallas import tpu as pltpu
from jax.experimental.pallas import tpu_sc as plsc

HIDDEN = 128
VOCAB = 30522
MAX_SEQ = 512
BATCH = 1024

N_TOK = BATCH * MAX_SEQ
NUM_WORKERS = 32
TOK_PER_W = N_TOK // NUM_WORKERS
CHUNK = 64
NCHUNK = TOK_PER_W // CHUNK
NBUF = 4
NSUPER = NCHUNK // NBUF
POS_PERIOD = MAX_SEQ // CHUNK
LANES = 16
PACKED = HIDDEN // 2
NBLK = HIDDEN // (2 * LANES)
SCALE = math.sqrt(HIDDEN)


def _pos_encoding(max_seq_len, hidden):
    pe = np.zeros((max_seq_len, hidden), dtype=np.float32)
    pos = np.arange(max_seq_len, dtype=np.float64)[:, None]
    i = np.arange(0, hidden, 2, dtype=np.float64)
    pe[:, 0::2] = np.sin(pos / (10000.0 ** (2.0 * i / hidden)))
    pe[:, 1::2] = np.cos(pos / (10000.0 ** (2.0 * (i + 1.0) / hidden)))
    return pe


_PE = _pos_encoding(MAX_SEQ, HIDDEN)


def _pack_bf16(x):
    n = x.shape[0]
    xb = x.reshape(n, NBLK, 2, LANES)
    a = lax.bitcast_convert_type(
        xb[:, :, 0, :].astype(jnp.bfloat16), jnp.uint16
    ).astype(jnp.uint32)
    b = lax.bitcast_convert_type(
        xb[:, :, 1, :].astype(jnp.bfloat16), jnp.uint16
    ).astype(jnp.uint32)
    return lax.bitcast_convert_type(a | (b << 16), jnp.int32).reshape(n, PACKED)


PACK_ROWS_PER_W = -(-VOCAB // NUM_WORKERS)
PACK_CHUNK = 53
PACK_NSTEP = -(-PACK_ROWS_PER_W // PACK_CHUNK)


@functools.partial(
    pl.kernel,
    out_type=jax.ShapeDtypeStruct((VOCAB, PACKED), jnp.int32),
    mesh=plsc.VectorSubcoreMesh(core_axis_name="c", subcore_axis_name="s"),
    compiler_params=pltpu.CompilerParams(
        needs_layout_passes=False, use_tc_tiling_on_sc=False
    ),
    scratch_types=[
        pltpu.VMEM((2, PACK_CHUNK, HIDDEN), jnp.float32),
        pltpu.VMEM((2, PACK_CHUNK, PACKED), jnp.int32),
        pltpu.SemaphoreType.DMA((2,)),
        pltpu.SemaphoreType.DMA((2,)),
    ],
)
def _pack_sc(tbl_hbm, out_hbm, raw_v, pk_v, gsem, wsem):
    wid = lax.axis_index("s") * 2 + lax.axis_index("c")
    base = wid * PACK_ROWS_PER_W

    def start_read(b, step):
        start = base + step * PACK_CHUNK
        start = jnp.minimum(start, VOCAB - PACK_CHUNK)
        pltpu.async_copy(
            tbl_hbm.at[pl.ds(start, PACK_CHUNK)], raw_v.at[b], gsem.at[b]
        )

    def wait_read(b):
        pltpu.make_async_copy(
            tbl_hbm.at[pl.ds(0, PACK_CHUNK)], raw_v.at[b], gsem.at[b]
        ).wait()

    def start_write(b, step):
        start = base + step * PACK_CHUNK
        start = jnp.minimum(start, VOCAB - PACK_CHUNK)
        pltpu.async_copy(
            pk_v.at[b], out_hbm.at[pl.ds(start, PACK_CHUNK)], wsem.at[b]
        )

    def wait_write(b):
        pltpu.make_async_copy(
            pk_v.at[b], out_hbm.at[pl.ds(0, PACK_CHUNK)], wsem.at[b]
        ).wait()

    def compute(b):
        @plsc.parallel_loop(0, PACK_CHUNK, 1, unroll=2)
        def _(j):
            for k in range(NBLK):
                lo = raw_v[b, j, pl.ds(2 * k * LANES, LANES)]
                hi = raw_v[b, j, pl.ds((2 * k + 1) * LANES, LANES)]
                w = plsc.pack(lo, hi, format=plsc.PackFormat.INTERLEAVED)
                pk_v[b, j, pl.ds(k * LANES, LANES)] = plsc.bitcast(w, jnp.int32)

    start_read(0, 0)
    start_read(1, 1)
    wait_read(0)
    compute(0)
    start_write(0, 0)
    wait_read(1)
    compute(1)
    start_write(1, 1)

    def step_body(s2, carry):
        for b in range(2):
            s = 2 * s2 + b
            wait_write(b)
            start_read(b, s)
            wait_read(b)
            compute(b)
            start_write(b, s)
        return carry

    lax.fori_loop(1, PACK_NSTEP // 2, step_body, 0)
    wait_write(0)
    wait_write(1)


@functools.partial(
    pl.kernel,
    out_type=jax.ShapeDtypeStruct((N_TOK, HIDDEN), jnp.float32),
    mesh=plsc.VectorSubcoreMesh(core_axis_name="c", subcore_axis_name="s"),
    compiler_params=pltpu.CompilerParams(
        needs_layout_passes=False, use_tc_tiling_on_sc=False
    ),
    scratch_types=[
        pltpu.VMEM((TOK_PER_W,), jnp.int32),
        pltpu.VMEM((MAX_SEQ, PACKED), jnp.int32),
        pltpu.VMEM((NBUF, CHUNK, PACKED), jnp.int32),
        pltpu.VMEM((NBUF, CHUNK, HIDDEN), jnp.float32),
        pltpu.SemaphoreType.DMA((NBUF,)),
        pltpu.SemaphoreType.DMA((NBUF,)),
    ],
)
def _encode(idx_hbm, tbl_hbm, pe_hbm, out_hbm,
            idx_v, pe_v, gath_v, out_v, gsem, wsem):
    wid = lax.axis_index("s") * 2 + lax.axis_index("c")
    base = wid * TOK_PER_W
    pltpu.sync_copy(pe_hbm, pe_v)
    pltpu.sync_copy(idx_hbm.at[pl.ds(base, TOK_PER_W)], idx_v)

    def start_gather(b, c):
        pltpu.async_copy(
            tbl_hbm.at[idx_v.at[pl.ds(c * CHUNK, CHUNK)]],
            gath_v.at[b],
            gsem.at[b],
        )

    def wait_gather(b):
        pltpu.make_async_copy(
            tbl_hbm.at[idx_v.at[pl.ds(0, CHUNK)]], gath_v.at[b], gsem.at[b]
        ).wait()

    def start_write(b, c):
        pltpu.async_copy(
            out_v.at[b], out_hbm.at[pl.ds(base + c * CHUNK, CHUNK)], wsem.at[b]
        )

    def wait_write(b):
        pltpu.make_async_copy(
            out_v.at[b], out_hbm.at[pl.ds(base, CHUNK)], wsem.at[b]
        ).wait()

    def compute(b, c):
        gbuf = gath_v.at[b]
        obuf = out_v.at[b]
        prow = (c % POS_PERIOD) * CHUNK

        @plsc.parallel_loop(0, CHUNK, 1, unroll=2)
        def _(j):
            for k in range(NBLK):
                sl = pl.ds(k * LANES, LANES)
                u = plsc.bitcast(gbuf[j, sl], jnp.bfloat16)
                p = plsc.bitcast(pe_v[prow + j, sl], jnp.bfloat16)
                r_lo, r_hi = plsc.unpack(u, format=plsc.PackFormat.INTERLEAVED)
                p_lo, p_hi = plsc.unpack(p, format=plsc.PackFormat.INTERLEAVED)
                obuf[j, pl.ds(2 * k * LANES, LANES)] = r_lo * SCALE + p_lo
                obuf[j, pl.ds((2 * k + 1) * LANES, LANES)] = r_hi * SCALE + p_hi

    for b in range(NBUF):
        start_gather(b, b)

    for b in range(NBUF):
        wait_gather(b)
        compute(b, b)
        start_gather(b, NBUF + b)
        start_write(b, b)

    def super_step(s, carry):
        for b in range(NBUF):
            c = s * NBUF + b
            wait_gather(b)
            wait_write(b)
            compute(b, c)
            start_gather(b, c + NBUF)
            start_write(b, c)
        return carry

    lax.fori_loop(1, NSUPER - 1, super_step, 0)

    for b in range(NBUF):
        c = (NSUPER - 1) * NBUF + b
        wait_gather(b)
        wait_write(b)
        compute(b, c)
        start_write(b, c)
    for b in range(NBUF):
        wait_write(b)


def kernel(text_batch, embed_table):
    b, l = text_batch.shape
    idx = text_batch.reshape(-1)
    tbl = _pack_sc(embed_table)
    pe = _pack_bf16(jnp.asarray(_PE))
    out = _encode(idx, tbl, pe)
    return out.reshape(b, l, HIDDEN)


# speedup vs baseline: 1.1159x; 1.0421x over previous
import functools
import math

import jax
import jax.numpy as jnp
import numpy as np
from jax import lax
from jax.experimental import pallas as pl
from jax.experimental.pallas import tpu as pltpu
from jax.experimental.pallas import tpu_sc as plsc

HIDDEN = 128
VOCAB = 30522
MAX_SEQ = 512
BATCH = 1024

N_TOK = BATCH * MAX_SEQ
NUM_WORKERS = 32
TOK_PER_W = N_TOK // NUM_WORKERS
CHUNK = 64
NCHUNK = TOK_PER_W // CHUNK
NBUF = 4
NSUPER = NCHUNK // NBUF
POS_PERIOD = MAX_SEQ // CHUNK
LANES = 16
PACKED = HIDDEN // 2
NBLK = HIDDEN // (2 * LANES)
SCALE = math.sqrt(HIDDEN)


def _pos_encoding(max_seq_len, hidden):
    pe = np.zeros((max_seq_len, hidden), dtype=np.float32)
    pos = np.arange(max_seq_len, dtype=np.float64)[:, None]
    i = np.arange(0, hidden, 2, dtype=np.float64)
    pe[:, 0::2] = np.sin(pos / (10000.0 ** (2.0 * i / hidden)))
    pe[:, 1::2] = np.cos(pos / (10000.0 ** (2.0 * (i + 1.0) / hidden)))
    return pe


_PE = _pos_encoding(MAX_SEQ, HIDDEN)


def _pack_bf16(x):
    n = x.shape[0]
    xb = x.reshape(n, NBLK, 2, LANES)
    a = lax.bitcast_convert_type(
        xb[:, :, 0, :].astype(jnp.bfloat16), jnp.uint16
    ).astype(jnp.uint32)
    b = lax.bitcast_convert_type(
        xb[:, :, 1, :].astype(jnp.bfloat16), jnp.uint16
    ).astype(jnp.uint32)
    return lax.bitcast_convert_type(a | (b << 16), jnp.int32).reshape(n, PACKED)


PACK_ROWS_PER_W = -(-VOCAB // NUM_WORKERS)
PACK_CHUNK = 53
PACK_NSTEP = -(-PACK_ROWS_PER_W // PACK_CHUNK)


@functools.partial(
    pl.kernel,
    out_type=jax.ShapeDtypeStruct((VOCAB, PACKED), jnp.int32),
    mesh=plsc.VectorSubcoreMesh(core_axis_name="c", subcore_axis_name="s"),
    compiler_params=pltpu.CompilerParams(
        needs_layout_passes=False, use_tc_tiling_on_sc=False
    ),
    scratch_types=[
        pltpu.VMEM((2, PACK_CHUNK, HIDDEN), jnp.float32),
        pltpu.VMEM((2, PACK_CHUNK, PACKED), jnp.int32),
        pltpu.SemaphoreType.DMA((2,)),
        pltpu.SemaphoreType.DMA((2,)),
    ],
)
def _pack_sc(tbl_hbm, out_hbm, raw_v, pk_v, gsem, wsem):
    wid = lax.axis_index("s") * 2 + lax.axis_index("c")
    base = wid * PACK_ROWS_PER_W

    def start_read(b, step):
        start = base + step * PACK_CHUNK
        start = jnp.minimum(start, VOCAB - PACK_CHUNK)
        pltpu.async_copy(
            tbl_hbm.at[pl.ds(start, PACK_CHUNK)], raw_v.at[b], gsem.at[b]
        )

    def wait_read(b):
        pltpu.make_async_copy(
            tbl_hbm.at[pl.ds(0, PACK_CHUNK)], raw_v.at[b], gsem.at[b]
        ).wait()

    def start_write(b, step):
        start = base + step * PACK_CHUNK
        start = jnp.minimum(start, VOCAB - PACK_CHUNK)
        pltpu.async_copy(
            pk_v.at[b], out_hbm.at[pl.ds(start, PACK_CHUNK)], wsem.at[b]
        )

    def wait_write(b):
        pltpu.make_async_copy(
            pk_v.at[b], out_hbm.at[pl.ds(0, PACK_CHUNK)], wsem.at[b]
        ).wait()

    def compute(b):
        @plsc.parallel_loop(0, PACK_CHUNK, 1, unroll=2)
        def _(j):
            for k in range(NBLK):
                lo = raw_v[b, j, pl.ds(2 * k * LANES, LANES)]
                hi = raw_v[b, j, pl.ds((2 * k + 1) * LANES, LANES)]
                w = plsc.pack(lo, hi, format=plsc.PackFormat.INTERLEAVED)
                pk_v[b, j, pl.ds(k * LANES, LANES)] = plsc.bitcast(w, jnp.int32)

    start_read(0, 0)
    start_read(1, 1)
    for b in range(2):
        wait_read(b)
        compute(b)
        start_write(b, b)
        start_read(b, 2 + b)

    def step_body(s2, carry):
        for b in range(2):
            s = 2 * s2 + b
            wait_read(b)
            wait_write(b)
            compute(b)
            start_write(b, s)
            start_read(b, s + 2)
        return carry

    lax.fori_loop(1, PACK_NSTEP // 2 - 1, step_body, 0)

    for b in range(2):
        s = PACK_NSTEP - 2 + b
        wait_read(b)
        wait_write(b)
        compute(b)
        start_write(b, s)
    for b in range(2):
        wait_write(b)


@functools.partial(
    pl.kernel,
    out_type=jax.ShapeDtypeStruct((N_TOK, HIDDEN), jnp.float32),
    mesh=plsc.VectorSubcoreMesh(core_axis_name="c", subcore_axis_name="s"),
    compiler_params=pltpu.CompilerParams(
        needs_layout_passes=False, use_tc_tiling_on_sc=False
    ),
    scratch_types=[
        pltpu.VMEM((TOK_PER_W,), jnp.int32),
        pltpu.VMEM((MAX_SEQ, PACKED), jnp.int32),
        pltpu.VMEM((NBUF, CHUNK, PACKED), jnp.int32),
        pltpu.VMEM((NBUF, CHUNK, HIDDEN), jnp.float32),
        pltpu.SemaphoreType.DMA((NBUF,)),
        pltpu.SemaphoreType.DMA((NBUF,)),
    ],
)
def _encode(idx_hbm, tbl_hbm, pe_hbm, out_hbm,
            idx_v, pe_v, gath_v, out_v, gsem, wsem):
    wid = lax.axis_index("s") * 2 + lax.axis_index("c")
    base = wid * TOK_PER_W
    pltpu.sync_copy(pe_hbm, pe_v)
    pltpu.sync_copy(idx_hbm.at[pl.ds(base, TOK_PER_W)], idx_v)

    def start_gather(b, c):
        pltpu.async_copy(
            tbl_hbm.at[idx_v.at[pl.ds(c * CHUNK, CHUNK)]],
            gath_v.at[b],
            gsem.at[b],
        )

    def wait_gather(b):
        pltpu.make_async_copy(
            tbl_hbm.at[idx_v.at[pl.ds(0, CHUNK)]], gath_v.at[b], gsem.at[b]
        ).wait()

    def start_write(b, c):
        pltpu.async_copy(
            out_v.at[b], out_hbm.at[pl.ds(base + c * CHUNK, CHUNK)], wsem.at[b]
        )

    def wait_write(b):
        pltpu.make_async_copy(
            out_v.at[b], out_hbm.at[pl.ds(base, CHUNK)], wsem.at[b]
        ).wait()

    def compute(b, c):
        gbuf = gath_v.at[b]
        obuf = out_v.at[b]
        prow = (c % POS_PERIOD) * CHUNK

        @plsc.parallel_loop(0, CHUNK, 1, unroll=2)
        def _(j):
            for k in range(NBLK):
                sl = pl.ds(k * LANES, LANES)
                u = plsc.bitcast(gbuf[j, sl], jnp.bfloat16)
                p = plsc.bitcast(pe_v[prow + j, sl], jnp.bfloat16)
                r_lo, r_hi = plsc.unpack(u, format=plsc.PackFormat.INTERLEAVED)
                p_lo, p_hi = plsc.unpack(p, format=plsc.PackFormat.INTERLEAVED)
                obuf[j, pl.ds(2 * k * LANES, LANES)] = r_lo * SCALE + p_lo
                obuf[j, pl.ds((2 * k + 1) * LANES, LANES)] = r_hi * SCALE + p_hi

    for b in range(NBUF):
        start_gather(b, b)

    for b in range(NBUF):
        wait_gather(b)
        compute(b, b)
        start_gather(b, NBUF + b)
        start_write(b, b)

    def super_step(s, carry):
        for b in range(NBUF):
            c = s * NBUF + b
            wait_gather(b)
            wait_write(b)
            compute(b, c)
            start_gather(b, c + NBUF)
            start_write(b, c)
        return carry

    lax.fori_loop(1, NSUPER - 1, super_step, 0)

    for b in range(NBUF):
        c = (NSUPER - 1) * NBUF + b
        wait_gather(b)
        wait_write(b)
        compute(b, c)
        start_write(b, c)
    for b in range(NBUF):
        wait_write(b)


def kernel(text_batch, embed_table):
    b, l = text_batch.shape
    idx = text_batch.reshape(-1)
    tbl = _pack_sc(embed_table)
    pe = _pack_bf16(jnp.asarray(_PE))
    out = _encode(idx, tbl, pe)
    return out.reshape(b, l, HIDDEN)


# speedup vs baseline: 1.1342x; 1.0164x over previous
import functools
import math

import jax
import jax.numpy as jnp
import numpy as np
from jax import lax
from jax.experimental import pallas as pl
from jax.experimental.pallas import tpu as pltpu
from jax.experimental.pallas import tpu_sc as plsc

HIDDEN = 128
VOCAB = 30522
MAX_SEQ = 512
BATCH = 1024

N_TOK = BATCH * MAX_SEQ
NUM_WORKERS = 32
TOK_PER_W = N_TOK // NUM_WORKERS
CHUNK = 128
NCHUNK = TOK_PER_W // CHUNK
NBUF = 2
NSUPER = NCHUNK // NBUF
POS_PERIOD = MAX_SEQ // CHUNK
LANES = 16
PACKED = HIDDEN // 2
NBLK = HIDDEN // (2 * LANES)
SCALE = math.sqrt(HIDDEN)


def _pos_encoding(max_seq_len, hidden):
    pe = np.zeros((max_seq_len, hidden), dtype=np.float32)
    pos = np.arange(max_seq_len, dtype=np.float64)[:, None]
    i = np.arange(0, hidden, 2, dtype=np.float64)
    pe[:, 0::2] = np.sin(pos / (10000.0 ** (2.0 * i / hidden)))
    pe[:, 1::2] = np.cos(pos / (10000.0 ** (2.0 * (i + 1.0) / hidden)))
    return pe


_PE = _pos_encoding(MAX_SEQ, HIDDEN)


def _pack_bf16(x):
    n = x.shape[0]
    xb = x.reshape(n, NBLK, 2, LANES)
    a = lax.bitcast_convert_type(
        xb[:, :, 0, :].astype(jnp.bfloat16), jnp.uint16
    ).astype(jnp.uint32)
    b = lax.bitcast_convert_type(
        xb[:, :, 1, :].astype(jnp.bfloat16), jnp.uint16
    ).astype(jnp.uint32)
    return lax.bitcast_convert_type(a | (b << 16), jnp.int32).reshape(n, PACKED)


PACK_ROWS_PER_W = -(-VOCAB // NUM_WORKERS)
PACK_CHUNK = 53
PACK_NSTEP = -(-PACK_ROWS_PER_W // PACK_CHUNK)


@functools.partial(
    pl.kernel,
    out_type=jax.ShapeDtypeStruct((VOCAB, PACKED), jnp.int32),
    mesh=plsc.VectorSubcoreMesh(core_axis_name="c", subcore_axis_name="s"),
    compiler_params=pltpu.CompilerParams(
        needs_layout_passes=False, use_tc_tiling_on_sc=False
    ),
    scratch_types=[
        pltpu.VMEM((2, PACK_CHUNK, HIDDEN), jnp.float32),
        pltpu.VMEM((2, PACK_CHUNK, PACKED), jnp.int32),
        pltpu.SemaphoreType.DMA((2,)),
        pltpu.SemaphoreType.DMA((2,)),
    ],
)
def _pack_sc(tbl_hbm, out_hbm, raw_v, pk_v, gsem, wsem):
    wid = lax.axis_index("s") * 2 + lax.axis_index("c")
    base = wid * PACK_ROWS_PER_W

    def start_read(b, step):
        start = base + step * PACK_CHUNK
        start = jnp.minimum(start, VOCAB - PACK_CHUNK)
        pltpu.async_copy(
            tbl_hbm.at[pl.ds(start, PACK_CHUNK)], raw_v.at[b], gsem.at[b]
        )

    def wait_read(b):
        pltpu.make_async_copy(
            tbl_hbm.at[pl.ds(0, PACK_CHUNK)], raw_v.at[b], gsem.at[b]
        ).wait()

    def start_write(b, step):
        start = base + step * PACK_CHUNK
        start = jnp.minimum(start, VOCAB - PACK_CHUNK)
        pltpu.async_copy(
            pk_v.at[b], out_hbm.at[pl.ds(start, PACK_CHUNK)], wsem.at[b]
        )

    def wait_write(b):
        pltpu.make_async_copy(
            pk_v.at[b], out_hbm.at[pl.ds(0, PACK_CHUNK)], wsem.at[b]
        ).wait()

    def compute(b):
        @plsc.parallel_loop(0, PACK_CHUNK, 1, unroll=2)
        def _(j):
            for k in range(NBLK):
                lo = raw_v[b, j, pl.ds(2 * k * LANES, LANES)]
                hi = raw_v[b, j, pl.ds((2 * k + 1) * LANES, LANES)]
                w = plsc.pack(lo, hi, format=plsc.PackFormat.INTERLEAVED)
                pk_v[b, j, pl.ds(k * LANES, LANES)] = plsc.bitcast(w, jnp.int32)

    start_read(0, 0)
    start_read(1, 1)
    for b in range(2):
        wait_read(b)
        compute(b)
        start_write(b, b)
        start_read(b, 2 + b)

    def step_body(s2, carry):
        for b in range(2):
            s = 2 * s2 + b
            wait_read(b)
            wait_write(b)
            compute(b)
            start_write(b, s)
            start_read(b, s + 2)
        return carry

    lax.fori_loop(1, PACK_NSTEP // 2 - 1, step_body, 0)

    for b in range(2):
        s = PACK_NSTEP - 2 + b
        wait_read(b)
        wait_write(b)
        compute(b)
        start_write(b, s)
    for b in range(2):
        wait_write(b)


@functools.partial(
    pl.kernel,
    out_type=jax.ShapeDtypeStruct((N_TOK, HIDDEN), jnp.float32),
    mesh=plsc.VectorSubcoreMesh(core_axis_name="c", subcore_axis_name="s"),
    compiler_params=pltpu.CompilerParams(
        needs_layout_passes=False, use_tc_tiling_on_sc=False
    ),
    scratch_types=[
        pltpu.VMEM((TOK_PER_W,), jnp.int32),
        pltpu.VMEM((MAX_SEQ, PACKED), jnp.int32),
        pltpu.VMEM((NBUF, CHUNK, PACKED), jnp.int32),
        pltpu.VMEM((NBUF, CHUNK, HIDDEN), jnp.float32),
        pltpu.SemaphoreType.DMA((NBUF,)),
        pltpu.SemaphoreType.DMA((NBUF,)),
    ],
)
def _encode(idx_hbm, tbl_hbm, pe_hbm, out_hbm,
            idx_v, pe_v, gath_v, out_v, gsem, wsem):
    wid = lax.axis_index("s") * 2 + lax.axis_index("c")
    base = wid * TOK_PER_W
    pltpu.sync_copy(pe_hbm, pe_v)
    pltpu.sync_copy(idx_hbm.at[pl.ds(base, TOK_PER_W)], idx_v)

    def start_gather(b, c):
        pltpu.async_copy(
            tbl_hbm.at[idx_v.at[pl.ds(c * CHUNK, CHUNK)]],
            gath_v.at[b],
            gsem.at[b],
        )

    def wait_gather(b):
        pltpu.make_async_copy(
            tbl_hbm.at[idx_v.at[pl.ds(0, CHUNK)]], gath_v.at[b], gsem.at[b]
        ).wait()

    def start_write(b, c):
        pltpu.async_copy(
            out_v.at[b], out_hbm.at[pl.ds(base + c * CHUNK, CHUNK)], wsem.at[b]
        )

    def wait_write(b):
        pltpu.make_async_copy(
            out_v.at[b], out_hbm.at[pl.ds(base, CHUNK)], wsem.at[b]
        ).wait()

    def compute(b, c):
        gbuf = gath_v.at[b]
        obuf = out_v.at[b]
        prow = (c % POS_PERIOD) * CHUNK

        @plsc.parallel_loop(0, CHUNK, 1, unroll=2)
        def _(j):
            for k in range(NBLK):
                sl = pl.ds(k * LANES, LANES)
                u = plsc.bitcast(gbuf[j, sl], jnp.bfloat16)
                p = plsc.bitcast(pe_v[prow + j, sl], jnp.bfloat16)
                r_lo, r_hi = plsc.unpack(u, format=plsc.PackFormat.INTERLEAVED)
                p_lo, p_hi = plsc.unpack(p, format=plsc.PackFormat.INTERLEAVED)
                obuf[j, pl.ds(2 * k * LANES, LANES)] = r_lo * SCALE + p_lo
                obuf[j, pl.ds((2 * k + 1) * LANES, LANES)] = r_hi * SCALE + p_hi

    for b in range(NBUF):
        start_gather(b, b)

    for b in range(NBUF):
        wait_gather(b)
        compute(b, b)
        start_gather(b, NBUF + b)
        start_write(b, b)

    def super_step(s, carry):
        for b in range(NBUF):
            c = s * NBUF + b
            wait_gather(b)
            wait_write(b)
            compute(b, c)
            start_gather(b, c + NBUF)
            start_write(b, c)
        return carry

    lax.fori_loop(1, NSUPER - 1, super_step, 0)

    for b in range(NBUF):
        c = (NSUPER - 1) * NBUF + b
        wait_gather(b)
        wait_write(b)
        compute(b, c)
        start_write(b, c)
    for b in range(NBUF):
        wait_write(b)


def kernel(text_batch, embed_table):
    b, l = text_batch.shape
    idx = text_batch.reshape(-1)
    tbl = _pack_sc(embed_table)
    pe = _pack_bf16(jnp.asarray(_PE))
    out = _encode(idx, tbl, pe)
    return out.reshape(b, l, HIDDEN)
